# baseline (speedup 1.0000x reference)
import functools

import jax
import jax.numpy as jnp
from jax import lax
from jax.experimental import pallas as pl
from jax.experimental.pallas import tpu as pltpu
from jax.experimental.pallas import tpu_sc as plsc

_LANES = 128


def _pick_ch(e: int, target: int) -> int:
    ch = _LANES
    for d in range(_LANES, target + 1, _LANES):
        if e % d == 0:
            ch = d
    return ch


def _zero_vmem(buf, nwords):
    def zbody(i, cc):
        buf[pl.ds(i * 16, 16)] = jnp.zeros((16,), jnp.float32)
        return cc
    lax.fori_loop(0, nwords // 16, zbody, 0)


def _sc_degree(col1d, ew1d, *, npad, e, ch):
    info = plsc.get_sparse_core_info()
    ncore, nsub = info.num_cores, info.num_subcores
    nw = ncore * nsub
    sl = npad // nsub
    nchunks = e // ch
    mesh = plsc.VectorSubcoreMesh(core_axis_name="c", subcore_axis_name="s")

    @functools.partial(
        pl.kernel,
        out_type=jax.ShapeDtypeStruct((ncore * npad,), jnp.float32),
        mesh=mesh,
        scratch_types=[
            pltpu.VMEM((ch,), jnp.int32),
            pltpu.VMEM((ch,), jnp.float32),
            pltpu.VMEM((sl,), jnp.float32),
            pltpu.VMEM_SHARED((npad,), jnp.float32),
        ],
    )
    def k(col_hbm, ew_hbm, out_hbm, cbuf, wbuf, tmp, deg_sp):
        c = lax.axis_index("c")
        s = lax.axis_index("s")
        w = c * nsub + s
        seg = pl.ds(s * sl, sl)
        _zero_vmem(tmp, sl)
        pltpu.sync_copy(tmp, deg_sp.at[seg])
        plsc.subcore_barrier()
        my_chunks = (nchunks - w + nw - 1) // nw

        def body(i, carry):
            e0 = (w + i * nw) * ch
            pltpu.sync_copy(col_hbm.at[pl.ds(e0, ch)], cbuf)
            pltpu.sync_copy(ew_hbm.at[pl.ds(e0, ch)], wbuf)
            pltpu.sync_copy(wbuf, deg_sp.at[cbuf], add=True)
            return carry

        lax.fori_loop(0, my_chunks, body, 0)
        plsc.subcore_barrier()
        pltpu.sync_copy(deg_sp.at[seg], tmp)
        pltpu.sync_copy(tmp, out_hbm.at[pl.ds(c * npad + s * sl, sl)])

    return k(col1d, ew1d)


def _sc_messages(row1d, col1d, ew1d, packed_pad, *, npad, e, ch):
    info = plsc.get_sparse_core_info()
    ncore, nsub = info.num_cores, info.num_subcores
    nw = ncore * nsub
    sl = npad // nsub
    nchunks = e // ch
    mesh = plsc.VectorSubcoreMesh(core_axis_name="c", subcore_axis_name="s")

    @functools.partial(
        pl.kernel,
        out_type=(
            jax.ShapeDtypeStruct((ncore * npad,), jnp.float32),
            jax.ShapeDtypeStruct((ncore * npad,), jnp.float32),
        ),
        mesh=mesh,
        compiler_params=pltpu.CompilerParams(needs_layout_passes=False),
        scratch_types=[
            pltpu.VMEM((ch,), jnp.int32),
            pltpu.VMEM((ch,), jnp.int32),
            pltpu.VMEM((ch,), jnp.float32),
            pltpu.VMEM((ch,), jnp.float32),
            pltpu.VMEM((ch,), jnp.float32),
            pltpu.VMEM((sl // 2,), jnp.float32),
            pltpu.VMEM((npad,), jnp.int32),
            pltpu.VMEM_SHARED((npad,), jnp.float32),
            pltpu.VMEM_SHARED((npad,), jnp.float32),
        ],
    )
    def k(row_hbm, col_hbm, ew_hbm, packed_hbm, s_out, u_out,
          rbuf, cbuf, wbuf, m1b, m2b, tmp, ptab, ssp, usp):
        c = lax.axis_index("c")
        s = lax.axis_index("s")
        w = c * nsub + s
        hw = sl // 2
        pltpu.sync_copy(packed_hbm, ptab)
        _zero_vmem(tmp, hw)
        for half in range(2):
            hseg = pl.ds(s * sl + half * hw, hw)
            pltpu.sync_copy(tmp, ssp.at[hseg])
            pltpu.sync_copy(tmp, usp.at[hseg])
        plsc.subcore_barrier()
        my_chunks = (nchunks - w + nw - 1) // nw

        def body(i, carry):
            e0 = (w + i * nw) * ch
            pltpu.sync_copy(row_hbm.at[pl.ds(e0, ch)], rbuf)
            pltpu.sync_copy(col_hbm.at[pl.ds(e0, ch)], cbuf)
            pltpu.sync_copy(ew_hbm.at[pl.ds(e0, ch)], wbuf)

            def cbody(j, cc):
                sl16 = pl.ds(j * 16, 16)
                ri = rbuf[sl16]
                ci = cbuf[sl16]
                gr = plsc.load_gather(ptab, [ri])
                gc = plsc.load_gather(ptab, [ci])
                a_row = plsc.bitcast(gr & jnp.int32(-65536), jnp.float32)
                dinv_col = plsc.bitcast(gc << 16, jnp.float32)
                wv = wbuf[sl16]
                m1b[sl16] = wv * a_row
                m2b[sl16] = wv * dinv_col
                return cc

            lax.fori_loop(0, ch // 16, cbody, 0)
            pltpu.sync_copy(m1b, ssp.at[cbuf], add=True)
            pltpu.sync_copy(m2b, usp.at[rbuf], add=True)
            return carry

        lax.fori_loop(0, my_chunks, body, 0)
        plsc.subcore_barrier()
        for half in range(2):
            hseg = pl.ds(s * sl + half * hw, hw)
            oseg = pl.ds(c * npad + s * sl + half * hw, hw)
            pltpu.sync_copy(ssp.at[hseg], tmp)
            pltpu.sync_copy(tmp, s_out.at[oseg])
            pltpu.sync_copy(usp.at[hseg], tmp)
            pltpu.sync_copy(tmp, u_out.at[oseg])

    return k(row1d, col1d, ew1d, packed_pad)


def _tc_prep(degp, xp, *, n, npad):
    rows = npad // _LANES

    def body(dp_ref, xp_ref, out_ref, pk_ref):
        deg = dp_ref[0] + dp_ref[1] + 1.0
        dinv = lax.rsqrt(deg)
        i0 = lax.broadcasted_iota(jnp.int32, (rows, _LANES), 0)
        i1 = lax.broadcasted_iota(jnp.int32, (rows, _LANES), 1)
        valid = (i0 * _LANES + i1) < n
        dinv = jnp.where(valid, dinv, 0.0)
        a = dinv * xp_ref[...]
        out_ref[0] = dinv
        out_ref[1] = a
        rb_a = lax.bitcast_convert_type(a, jnp.uint32) + jnp.uint32(0x8000)
        rb_d = lax.bitcast_convert_type(dinv, jnp.uint32) + jnp.uint32(0x8000)
        packed = (rb_a & jnp.uint32(0xFFFF0000)) | (rb_d >> jnp.uint32(16))
        pk_ref[...] = lax.bitcast_convert_type(packed, jnp.int32)

    return pl.pallas_call(
        body,
        out_shape=(
            jax.ShapeDtypeStruct((2, rows, _LANES), jnp.float32),
            jax.ShapeDtypeStruct((rows, _LANES), jnp.int32),
        ),
    )(degp, xp)


def _tc_tail(sp, up, da, W1, b1, W2, b2, *, n, npad):
    rows = npad // _LANES
    f1 = W1.shape[1]
    f2 = W2.shape[1]

    def body(sp_ref, up_ref, da_ref, w1_ref, b1_ref, w2_ref, b2_ref, out_ref):
        dinv = da_ref[0]
        a = da_ref[1]
        s1 = dinv * (sp_ref[0] + sp_ref[1] + a)
        t = dinv * (up_ref[0] + up_ref[1] + dinv)
        vs = []
        for j in range(f1):
            hj = jnp.maximum(s1 * w1_ref[0, j] + b1_ref[j], 0.0)
            vs.append(jnp.sum(t * hj))
        zs = []
        for kk in range(f2):
            zs.append(sum(vs[j] * w2_ref[j, kk] for j in range(f1))
                      + float(n) * b2_ref[kk])
        m = zs[0]
        for kk in range(1, f2):
            m = jnp.maximum(m, zs[kk])
        es = [jnp.exp(z - m) for z in zs]
        tot = es[0]
        for kk in range(1, f2):
            tot = tot + es[kk]
        inv = 1.0 / tot
        colv = lax.broadcasted_iota(jnp.int32, (1, f2), 1)
        acc = jnp.where(colv == 0, es[0] * inv, 0.0)
        for kk in range(1, f2):
            acc = jnp.where(colv == kk, es[kk] * inv, acc)
        out_ref[...] = acc

    smemspec = pl.BlockSpec(memory_space=pltpu.SMEM)
    return pl.pallas_call(
        body,
        in_specs=[
            pl.BlockSpec((2, rows, _LANES), lambda: (0, 0, 0)),
            pl.BlockSpec((2, rows, _LANES), lambda: (0, 0, 0)),
            pl.BlockSpec((2, rows, _LANES), lambda: (0, 0, 0)),
            smemspec, smemspec, smemspec, smemspec,
        ],
        out_shape=jax.ShapeDtypeStruct((1, f2), jnp.float32),
    )(sp, up, da, W1, b1, W2, b2)


def kernel(x, edge_index, edge_attr, W1, b1, W2, b2):
    n = x.shape[0]
    e = edge_index.shape[1]
    npad = ((n + _LANES - 1) // _LANES) * _LANES

    row1d = edge_index[0]
    col1d = edge_index[1]
    xp = jnp.pad(x[:, 0], (0, npad - n)).reshape(npad // _LANES, _LANES)

    ch1 = _pick_ch(e, 25600)
    ch2 = _pick_ch(e, 2560)

    degp = _sc_degree(col1d, ew1d := edge_attr, npad=npad, e=e, ch=ch1)
    da, packed = _tc_prep(degp.reshape(2, npad // _LANES, _LANES), xp,
                          n=n, npad=npad)
    s_p, u_p = _sc_messages(row1d, col1d, ew1d, packed.reshape(npad),
                            npad=npad, e=e, ch=ch2)
    out = _tc_tail(s_p.reshape(2, npad // _LANES, _LANES),
                   u_p.reshape(2, npad // _LANES, _LANES),
                   da, W1, b1, W2, b2, n=n, npad=npad)
    return out


# speedup vs baseline: 1.0901x; 1.0901x over previous
import functools

import jax
import jax.numpy as jnp
from jax import lax
from jax.experimental import pallas as pl
from jax.experimental.pallas import tpu as pltpu
from jax.experimental.pallas import tpu_sc as plsc

_LANES = 128


def _pick_ch(e: int, target: int) -> int:
    ch = _LANES
    for d in range(_LANES, target + 1, _LANES):
        if e % d == 0:
            ch = d
    return ch


def _zero_vmem(buf, nwords):
    def zbody(i, cc):
        buf[pl.ds(i * 16, 16)] = jnp.zeros((16,), jnp.float32)
        return cc
    lax.fori_loop(0, nwords // 16, zbody, 0)


def _sc_degree(col1d, ew1d, *, npad, e, ch):
    info = plsc.get_sparse_core_info()
    ncore, nsub = info.num_cores, info.num_subcores
    nw = ncore * nsub
    sl = npad // nsub
    nchunks = e // ch
    mesh = plsc.VectorSubcoreMesh(core_axis_name="c", subcore_axis_name="s")

    @functools.partial(
        pl.kernel,
        out_type=jax.ShapeDtypeStruct((ncore * npad,), jnp.float32),
        mesh=mesh,
        scratch_types=[
            pltpu.VMEM((ch,), jnp.int32),
            pltpu.VMEM((ch,), jnp.float32),
            pltpu.VMEM((sl,), jnp.float32),
            pltpu.VMEM_SHARED((npad,), jnp.float32),
        ],
    )
    def k(col_hbm, ew_hbm, out_hbm, cbuf, wbuf, tmp, deg_sp):
        c = lax.axis_index("c")
        s = lax.axis_index("s")
        w = c * nsub + s
        seg = pl.ds(s * sl, sl)
        _zero_vmem(tmp, sl)
        pltpu.sync_copy(tmp, deg_sp.at[seg])
        plsc.subcore_barrier()
        my_chunks = (nchunks - w + nw - 1) // nw

        def body(i, carry):
            e0 = (w + i * nw) * ch
            pltpu.sync_copy(col_hbm.at[pl.ds(e0, ch)], cbuf)
            pltpu.sync_copy(ew_hbm.at[pl.ds(e0, ch)], wbuf)
            pltpu.sync_copy(wbuf, deg_sp.at[cbuf], add=True)
            return carry

        lax.fori_loop(0, my_chunks, body, 0)
        plsc.subcore_barrier()
        pltpu.sync_copy(deg_sp.at[seg], tmp)
        pltpu.sync_copy(tmp, out_hbm.at[pl.ds(c * npad + s * sl, sl)])

    return k(col1d, ew1d)


def _sc_messages(row1d, col1d, ew1d, packed_pad, *, npad, e, ch):
    info = plsc.get_sparse_core_info()
    ncore, nsub = info.num_cores, info.num_subcores
    nw = ncore * nsub
    sl = npad // nsub
    nchunks = e // ch
    mesh = plsc.VectorSubcoreMesh(core_axis_name="c", subcore_axis_name="s")

    @functools.partial(
        pl.kernel,
        out_type=(
            jax.ShapeDtypeStruct((ncore * npad,), jnp.float32),
            jax.ShapeDtypeStruct((ncore * npad,), jnp.float32),
        ),
        mesh=mesh,
        compiler_params=pltpu.CompilerParams(needs_layout_passes=False),
        scratch_types=[
            pltpu.VMEM((ch,), jnp.int32),
            pltpu.VMEM((ch,), jnp.int32),
            pltpu.VMEM((ch,), jnp.float32),
            pltpu.VMEM((ch,), jnp.float32),
            pltpu.VMEM((ch,), jnp.float32),
            pltpu.VMEM((sl // 2,), jnp.float32),
            pltpu.VMEM((npad,), jnp.int32),
            pltpu.VMEM_SHARED((npad,), jnp.float32),
            pltpu.VMEM_SHARED((npad,), jnp.float32),
        ],
    )
    def k(row_hbm, col_hbm, ew_hbm, packed_hbm, s_out, u_out,
          rbuf, cbuf, wbuf, m1b, m2b, tmp, ptab, ssp, usp):
        c = lax.axis_index("c")
        s = lax.axis_index("s")
        w = c * nsub + s
        hw = sl // 2
        pltpu.sync_copy(packed_hbm, ptab)
        _zero_vmem(tmp, hw)
        for half in range(2):
            hseg = pl.ds(s * sl + half * hw, hw)
            pltpu.sync_copy(tmp, ssp.at[hseg])
            pltpu.sync_copy(tmp, usp.at[hseg])
        plsc.subcore_barrier()
        my_chunks = (nchunks - w + nw - 1) // nw

        def body(i, carry):
            e0 = (w + i * nw) * ch
            pltpu.sync_copy(row_hbm.at[pl.ds(e0, ch)], rbuf)
            pltpu.sync_copy(col_hbm.at[pl.ds(e0, ch)], cbuf)
            pltpu.sync_copy(ew_hbm.at[pl.ds(e0, ch)], wbuf)

            @plsc.parallel_loop(0, ch // 16, unroll=4)
            def cbody(j):
                sl16 = pl.ds(j * 16, 16)
                ri = rbuf[sl16]
                ci = cbuf[sl16]
                gr = plsc.load_gather(ptab, [ri])
                gc = plsc.load_gather(ptab, [ci])
                a_row = plsc.bitcast(gr & jnp.int32(-65536), jnp.float32)
                dinv_col = plsc.bitcast(gc << 16, jnp.float32)
                wv = wbuf[sl16]
                m1b[sl16] = wv * a_row
                m2b[sl16] = wv * dinv_col
            pltpu.sync_copy(m1b, ssp.at[cbuf], add=True)
            pltpu.sync_copy(m2b, usp.at[rbuf], add=True)
            return carry

        lax.fori_loop(0, my_chunks, body, 0)
        plsc.subcore_barrier()
        for half in range(2):
            hseg = pl.ds(s * sl + half * hw, hw)
            oseg = pl.ds(c * npad + s * sl + half * hw, hw)
            pltpu.sync_copy(ssp.at[hseg], tmp)
            pltpu.sync_copy(tmp, s_out.at[oseg])
            pltpu.sync_copy(usp.at[hseg], tmp)
            pltpu.sync_copy(tmp, u_out.at[oseg])

    return k(row1d, col1d, ew1d, packed_pad)


def _tc_prep(degp, xp, *, n, npad):
    rows = npad // _LANES

    def body(dp_ref, xp_ref, out_ref, pk_ref):
        deg = dp_ref[0] + dp_ref[1] + 1.0
        dinv = lax.rsqrt(deg)
        i0 = lax.broadcasted_iota(jnp.int32, (rows, _LANES), 0)
        i1 = lax.broadcasted_iota(jnp.int32, (rows, _LANES), 1)
        valid = (i0 * _LANES + i1) < n
        dinv = jnp.where(valid, dinv, 0.0)
        a = dinv * xp_ref[...]
        out_ref[0] = dinv
        out_ref[1] = a
        rb_a = lax.bitcast_convert_type(a, jnp.uint32) + jnp.uint32(0x8000)
        rb_d = lax.bitcast_convert_type(dinv, jnp.uint32) + jnp.uint32(0x8000)
        packed = (rb_a & jnp.uint32(0xFFFF0000)) | (rb_d >> jnp.uint32(16))
        pk_ref[...] = lax.bitcast_convert_type(packed, jnp.int32)

    return pl.pallas_call(
        body,
        out_shape=(
            jax.ShapeDtypeStruct((2, rows, _LANES), jnp.float32),
            jax.ShapeDtypeStruct((rows, _LANES), jnp.int32),
        ),
    )(degp, xp)


def _tc_tail(sp, up, da, W1, b1, W2, b2, *, n, npad):
    rows = npad // _LANES
    f1 = W1.shape[1]
    f2 = W2.shape[1]

    def body(sp_ref, up_ref, da_ref, w1_ref, b1_ref, w2_ref, b2_ref, out_ref):
        dinv = da_ref[0]
        a = da_ref[1]
        s1 = dinv * (sp_ref[0] + sp_ref[1] + a)
        t = dinv * (up_ref[0] + up_ref[1] + dinv)
        vs = []
        for j in range(f1):
            hj = jnp.maximum(s1 * w1_ref[0, j] + b1_ref[j], 0.0)
            vs.append(jnp.sum(t * hj))
        zs = []
        for kk in range(f2):
            zs.append(sum(vs[j] * w2_ref[j, kk] for j in range(f1))
                      + float(n) * b2_ref[kk])
        m = zs[0]
        for kk in range(1, f2):
            m = jnp.maximum(m, zs[kk])
        es = [jnp.exp(z - m) for z in zs]
        tot = es[0]
        for kk in range(1, f2):
            tot = tot + es[kk]
        inv = 1.0 / tot
        colv = lax.broadcasted_iota(jnp.int32, (1, f2), 1)
        acc = jnp.where(colv == 0, es[0] * inv, 0.0)
        for kk in range(1, f2):
            acc = jnp.where(colv == kk, es[kk] * inv, acc)
        out_ref[...] = acc

    smemspec = pl.BlockSpec(memory_space=pltpu.SMEM)
    return pl.pallas_call(
        body,
        in_specs=[
            pl.BlockSpec((2, rows, _LANES), lambda: (0, 0, 0)),
            pl.BlockSpec((2, rows, _LANES), lambda: (0, 0, 0)),
            pl.BlockSpec((2, rows, _LANES), lambda: (0, 0, 0)),
            smemspec, smemspec, smemspec, smemspec,
        ],
        out_shape=jax.ShapeDtypeStruct((1, f2), jnp.float32),
    )(sp, up, da, W1, b1, W2, b2)


def kernel(x, edge_index, edge_attr, W1, b1, W2, b2):
    n = x.shape[0]
    e = edge_index.shape[1]
    npad = ((n + _LANES - 1) // _LANES) * _LANES

    row1d = edge_index[0]
    col1d = edge_index[1]
    xp = jnp.pad(x[:, 0], (0, npad - n)).reshape(npad // _LANES, _LANES)

    ch1 = _pick_ch(e, 25600)
    ch2 = _pick_ch(e, 2560)

    degp = _sc_degree(col1d, ew1d := edge_attr, npad=npad, e=e, ch=ch1)
    da, packed = _tc_prep(degp.reshape(2, npad // _LANES, _LANES), xp,
                          n=n, npad=npad)
    s_p, u_p = _sc_messages(row1d, col1d, ew1d, packed.reshape(npad),
                            npad=npad, e=e, ch=ch2)
    out = _tc_tail(s_p.reshape(2, npad // _LANES, _LANES),
                   u_p.reshape(2, npad // _LANES, _LANES),
                   da, W1, b1, W2, b2, n=n, npad=npad)
    return out


# speedup vs baseline: 1.0912x; 1.0011x over previous
import functools

import jax
import jax.numpy as jnp
from jax import lax
from jax.experimental import pallas as pl
from jax.experimental.pallas import tpu as pltpu
from jax.experimental.pallas import tpu_sc as plsc

_LANES = 128


def _pick_ch(e: int, target: int) -> int:
    ch = _LANES
    for d in range(_LANES, target + 1, _LANES):
        if e % d == 0:
            ch = d
    return ch


def _zero_vmem(buf, nwords):
    def zbody(i, cc):
        buf[pl.ds(i * 16, 16)] = jnp.zeros((16,), jnp.float32)
        return cc
    lax.fori_loop(0, nwords // 16, zbody, 0)


def _sc_degree(col1d, ew1d, *, npad, e, ch):
    info = plsc.get_sparse_core_info()
    ncore, nsub = info.num_cores, info.num_subcores
    nw = ncore * nsub
    sl = npad // nsub
    nchunks = e // ch
    mesh = plsc.VectorSubcoreMesh(core_axis_name="c", subcore_axis_name="s")

    @functools.partial(
        pl.kernel,
        out_type=jax.ShapeDtypeStruct((ncore * npad,), jnp.float32),
        mesh=mesh,
        scratch_types=[
            pltpu.VMEM((ch,), jnp.int32),
            pltpu.VMEM((ch,), jnp.float32),
            pltpu.VMEM((sl,), jnp.float32),
            pltpu.VMEM_SHARED((npad,), jnp.float32),
        ],
    )
    def k(col_hbm, ew_hbm, out_hbm, cbuf, wbuf, tmp, deg_sp):
        c = lax.axis_index("c")
        s = lax.axis_index("s")
        w = c * nsub + s
        seg = pl.ds(s * sl, sl)
        _zero_vmem(tmp, sl)
        pltpu.sync_copy(tmp, deg_sp.at[seg])
        plsc.subcore_barrier()
        my_chunks = (nchunks - w + nw - 1) // nw

        def body(i, carry):
            e0 = (w + i * nw) * ch
            pltpu.sync_copy(col_hbm.at[pl.ds(e0, ch)], cbuf)
            pltpu.sync_copy(ew_hbm.at[pl.ds(e0, ch)], wbuf)
            pltpu.sync_copy(wbuf, deg_sp.at[cbuf], add=True)
            return carry

        lax.fori_loop(0, my_chunks, body, 0)
        plsc.subcore_barrier()
        pltpu.sync_copy(deg_sp.at[seg], tmp)
        pltpu.sync_copy(tmp, out_hbm.at[pl.ds(c * npad + s * sl, sl)])

    return k(col1d, ew1d)


def _sc_messages(row1d, col1d, ew1d, packed_pad, *, npad, e, ch):
    info = plsc.get_sparse_core_info()
    ncore, nsub = info.num_cores, info.num_subcores
    nw = ncore * nsub
    sl = npad // nsub
    nchunks = e // ch
    mesh = plsc.VectorSubcoreMesh(core_axis_name="c", subcore_axis_name="s")

    @functools.partial(
        pl.kernel,
        out_type=(
            jax.ShapeDtypeStruct((ncore * npad,), jnp.float32),
            jax.ShapeDtypeStruct((ncore * npad,), jnp.float32),
        ),
        mesh=mesh,
        compiler_params=pltpu.CompilerParams(needs_layout_passes=False),
        scratch_types=[
            pltpu.VMEM((ch,), jnp.int32),
            pltpu.VMEM((ch,), jnp.int32),
            pltpu.VMEM((ch,), jnp.float32),
            pltpu.VMEM((ch,), jnp.float32),
            pltpu.VMEM((ch,), jnp.float32),
            pltpu.VMEM((sl // 2,), jnp.float32),
            pltpu.VMEM((npad,), jnp.int32),
            pltpu.VMEM_SHARED((npad,), jnp.float32),
            pltpu.VMEM_SHARED((npad,), jnp.float32),
        ],
    )
    def k(row_hbm, col_hbm, ew_hbm, packed_hbm, s_out, u_out,
          rbuf, cbuf, wbuf, m1b, m2b, tmp, ptab, ssp, usp):
        c = lax.axis_index("c")
        s = lax.axis_index("s")
        w = c * nsub + s
        hw = sl // 2
        pltpu.sync_copy(packed_hbm, ptab)
        _zero_vmem(tmp, hw)
        for half in range(2):
            hseg = pl.ds(s * sl + half * hw, hw)
            pltpu.sync_copy(tmp, ssp.at[hseg])
            pltpu.sync_copy(tmp, usp.at[hseg])
        plsc.subcore_barrier()
        my_chunks = (nchunks - w + nw - 1) // nw

        def body(i, carry):
            e0 = (w + i * nw) * ch
            pltpu.sync_copy(row_hbm.at[pl.ds(e0, ch)], rbuf)
            pltpu.sync_copy(col_hbm.at[pl.ds(e0, ch)], cbuf)
            pltpu.sync_copy(ew_hbm.at[pl.ds(e0, ch)], wbuf)

            @plsc.parallel_loop(0, ch // 16, unroll=8)
            def cbody(j):
                sl16 = pl.ds(j * 16, 16)
                ri = rbuf[sl16]
                ci = cbuf[sl16]
                gr = plsc.load_gather(ptab, [ri])
                gc = plsc.load_gather(ptab, [ci])
                a_row = plsc.bitcast(gr & jnp.int32(-65536), jnp.float32)
                dinv_col = plsc.bitcast(gc << 16, jnp.float32)
                wv = wbuf[sl16]
                m1b[sl16] = wv * a_row
                m2b[sl16] = wv * dinv_col
            pltpu.sync_copy(m1b, ssp.at[cbuf], add=True)
            pltpu.sync_copy(m2b, usp.at[rbuf], add=True)
            return carry

        lax.fori_loop(0, my_chunks, body, 0)
        plsc.subcore_barrier()
        for half in range(2):
            hseg = pl.ds(s * sl + half * hw, hw)
            oseg = pl.ds(c * npad + s * sl + half * hw, hw)
            pltpu.sync_copy(ssp.at[hseg], tmp)
            pltpu.sync_copy(tmp, s_out.at[oseg])
            pltpu.sync_copy(usp.at[hseg], tmp)
            pltpu.sync_copy(tmp, u_out.at[oseg])

    return k(row1d, col1d, ew1d, packed_pad)


def _tc_prep(degp, xp, *, n, npad):
    rows = npad // _LANES

    def body(dp_ref, xp_ref, out_ref, pk_ref):
        deg = dp_ref[0] + dp_ref[1] + 1.0
        dinv = lax.rsqrt(deg)
        i0 = lax.broadcasted_iota(jnp.int32, (rows, _LANES), 0)
        i1 = lax.broadcasted_iota(jnp.int32, (rows, _LANES), 1)
        valid = (i0 * _LANES + i1) < n
        dinv = jnp.where(valid, dinv, 0.0)
        a = dinv * xp_ref[...]
        out_ref[0] = dinv
        out_ref[1] = a
        rb_a = lax.bitcast_convert_type(a, jnp.uint32) + jnp.uint32(0x8000)
        rb_d = lax.bitcast_convert_type(dinv, jnp.uint32) + jnp.uint32(0x8000)
        packed = (rb_a & jnp.uint32(0xFFFF0000)) | (rb_d >> jnp.uint32(16))
        pk_ref[...] = lax.bitcast_convert_type(packed, jnp.int32)

    return pl.pallas_call(
        body,
        out_shape=(
            jax.ShapeDtypeStruct((2, rows, _LANES), jnp.float32),
            jax.ShapeDtypeStruct((rows, _LANES), jnp.int32),
        ),
    )(degp, xp)


def _tc_tail(sp, up, da, W1, b1, W2, b2, *, n, npad):
    rows = npad // _LANES
    f1 = W1.shape[1]
    f2 = W2.shape[1]

    def body(sp_ref, up_ref, da_ref, w1_ref, b1_ref, w2_ref, b2_ref, out_ref):
        dinv = da_ref[0]
        a = da_ref[1]
        s1 = dinv * (sp_ref[0] + sp_ref[1] + a)
        t = dinv * (up_ref[0] + up_ref[1] + dinv)
        vs = []
        for j in range(f1):
            hj = jnp.maximum(s1 * w1_ref[0, j] + b1_ref[j], 0.0)
            vs.append(jnp.sum(t * hj))
        zs = []
        for kk in range(f2):
            zs.append(sum(vs[j] * w2_ref[j, kk] for j in range(f1))
                      + float(n) * b2_ref[kk])
        m = zs[0]
        for kk in range(1, f2):
            m = jnp.maximum(m, zs[kk])
        es = [jnp.exp(z - m) for z in zs]
        tot = es[0]
        for kk in range(1, f2):
            tot = tot + es[kk]
        inv = 1.0 / tot
        colv = lax.broadcasted_iota(jnp.int32, (1, f2), 1)
        acc = jnp.where(colv == 0, es[0] * inv, 0.0)
        for kk in range(1, f2):
            acc = jnp.where(colv == kk, es[kk] * inv, acc)
        out_ref[...] = acc

    smemspec = pl.BlockSpec(memory_space=pltpu.SMEM)
    return pl.pallas_call(
        body,
        in_specs=[
            pl.BlockSpec((2, rows, _LANES), lambda: (0, 0, 0)),
            pl.BlockSpec((2, rows, _LANES), lambda: (0, 0, 0)),
            pl.BlockSpec((2, rows, _LANES), lambda: (0, 0, 0)),
            smemspec, smemspec, smemspec, smemspec,
        ],
        out_shape=jax.ShapeDtypeStruct((1, f2), jnp.float32),
    )(sp, up, da, W1, b1, W2, b2)


def kernel(x, edge_index, edge_attr, W1, b1, W2, b2):
    n = x.shape[0]
    e = edge_index.shape[1]
    npad = ((n + _LANES - 1) // _LANES) * _LANES

    row1d = edge_index[0]
    col1d = edge_index[1]
    xp = jnp.pad(x[:, 0], (0, npad - n)).reshape(npad // _LANES, _LANES)

    ch1 = _pick_ch(e, 25600)
    ch2 = _pick_ch(e, 2560)

    degp = _sc_degree(col1d, ew1d := edge_attr, npad=npad, e=e, ch=ch1)
    da, packed = _tc_prep(degp.reshape(2, npad // _LANES, _LANES), xp,
                          n=n, npad=npad)
    s_p, u_p = _sc_messages(row1d, col1d, ew1d, packed.reshape(npad),
                            npad=npad, e=e, ch=ch2)
    out = _tc_tail(s_p.reshape(2, npad // _LANES, _LANES),
                   u_p.reshape(2, npad // _LANES, _LANES),
                   da, W1, b1, W2, b2, n=n, npad=npad)
    return out


# speedup vs baseline: 1.3394x; 1.2274x over previous
import functools

import jax
import jax.numpy as jnp
from jax import lax
from jax.experimental import pallas as pl
from jax.experimental.pallas import tpu as pltpu
from jax.experimental.pallas import tpu_sc as plsc

_LANES = 128


def _pick_ch(e: int, target: int) -> int:
    ch = _LANES
    for d in range(_LANES, target + 1, _LANES):
        if e % d == 0:
            ch = d
    return ch


def _zero_vmem(buf, nwords):
    def zbody(i, cc):
        buf[pl.ds(i * 16, 16)] = jnp.zeros((16,), jnp.float32)
        return cc
    lax.fori_loop(0, nwords // 16, zbody, 0)


def _sc_degree(col1d, ew1d, *, npad, e, ch):
    info = plsc.get_sparse_core_info()
    ncore, nsub = info.num_cores, info.num_subcores
    nw = ncore * nsub
    sl = npad // nsub
    nchunks = e // ch
    mesh = plsc.VectorSubcoreMesh(core_axis_name="c", subcore_axis_name="s")

    @functools.partial(
        pl.kernel,
        out_type=jax.ShapeDtypeStruct((ncore * npad,), jnp.float32),
        mesh=mesh,
        scratch_types=[
            pltpu.VMEM((ch,), jnp.int32),
            pltpu.VMEM((ch,), jnp.float32),
            pltpu.VMEM((sl,), jnp.float32),
            pltpu.VMEM_SHARED((npad,), jnp.float32),
        ],
    )
    def k(col_hbm, ew_hbm, out_hbm, cbuf, wbuf, tmp, deg_sp):
        c = lax.axis_index("c")
        s = lax.axis_index("s")
        w = c * nsub + s
        seg = pl.ds(s * sl, sl)
        _zero_vmem(tmp, sl)
        pltpu.sync_copy(tmp, deg_sp.at[seg])
        plsc.subcore_barrier()
        my_chunks = (nchunks - w + nw - 1) // nw

        def body(i, carry):
            e0 = (w + i * nw) * ch
            pltpu.sync_copy(col_hbm.at[pl.ds(e0, ch)], cbuf)
            pltpu.sync_copy(ew_hbm.at[pl.ds(e0, ch)], wbuf)
            pltpu.sync_copy(wbuf, deg_sp.at[cbuf], add=True)
            return carry

        lax.fori_loop(0, my_chunks, body, 0)
        plsc.subcore_barrier()
        pltpu.sync_copy(deg_sp.at[seg], tmp)
        pltpu.sync_copy(tmp, out_hbm.at[pl.ds(c * npad + s * sl, sl)])

    return k(col1d, ew1d)


def _sc_messages(row1d, col1d, ew1d, packed_pad, *, npad, e, ch):
    info = plsc.get_sparse_core_info()
    ncore, nsub = info.num_cores, info.num_subcores
    nw = ncore * nsub
    sl = npad // nsub
    nchunks = e // ch
    mesh = plsc.VectorSubcoreMesh(core_axis_name="c", subcore_axis_name="s")

    @functools.partial(
        pl.kernel,
        out_type=(
            jax.ShapeDtypeStruct((ncore * npad,), jnp.float32),
            jax.ShapeDtypeStruct((ncore * npad,), jnp.float32),
        ),
        mesh=mesh,
        compiler_params=pltpu.CompilerParams(needs_layout_passes=False),
        scratch_types=[
            pltpu.VMEM((ch,), jnp.int32),
            pltpu.VMEM((ch,), jnp.int32),
            pltpu.VMEM((ch,), jnp.float32),
            pltpu.VMEM((ch,), jnp.float32),
            pltpu.VMEM((ch,), jnp.float32),
            pltpu.VMEM((sl // 2,), jnp.float32),
            pltpu.VMEM((npad,), jnp.int32),
            pltpu.VMEM_SHARED((npad,), jnp.float32),
            pltpu.VMEM_SHARED((npad,), jnp.float32),
            pltpu.SemaphoreType.DMA,
            pltpu.SemaphoreType.DMA,
            pltpu.SemaphoreType.DMA,
            pltpu.SemaphoreType.DMA,
            pltpu.SemaphoreType.DMA,
        ],
    )
    def k(row_hbm, col_hbm, ew_hbm, packed_hbm, s_out, u_out,
          rbuf, cbuf, wbuf, m1b, m2b, tmp, ptab, ssp, usp,
          sem_r, sem_c, sem_w, sem_s1, sem_s2):
        c = lax.axis_index("c")
        s = lax.axis_index("s")
        w = c * nsub + s
        hw = sl // 2
        pltpu.sync_copy(packed_hbm, ptab)
        _zero_vmem(tmp, hw)
        for half in range(2):
            hseg = pl.ds(s * sl + half * hw, hw)
            pltpu.sync_copy(tmp, ssp.at[hseg])
            pltpu.sync_copy(tmp, usp.at[hseg])
        plsc.subcore_barrier()
        my_chunks = (nchunks - w + nw - 1) // nw

        def body(i, carry):
            e0 = (w + i * nw) * ch
            hr = pltpu.async_copy(row_hbm.at[pl.ds(e0, ch)], rbuf, sem_r)
            hc = pltpu.async_copy(col_hbm.at[pl.ds(e0, ch)], cbuf, sem_c)
            hw2 = pltpu.async_copy(ew_hbm.at[pl.ds(e0, ch)], wbuf, sem_w)
            hr.wait()
            hc.wait()
            hw2.wait()

            @plsc.parallel_loop(0, ch // 16, unroll=8)
            def cbody(j):
                sl16 = pl.ds(j * 16, 16)
                ri = rbuf[sl16]
                ci = cbuf[sl16]
                gr = plsc.load_gather(ptab, [ri])
                gc = plsc.load_gather(ptab, [ci])
                a_row = plsc.bitcast(gr & jnp.int32(-65536), jnp.float32)
                dinv_col = plsc.bitcast(gc << 16, jnp.float32)
                wv = wbuf[sl16]
                m1b[sl16] = wv * a_row
                m2b[sl16] = wv * dinv_col
            h1 = pltpu.async_copy(m1b, ssp.at[cbuf], sem_s1, add=True)
            h2 = pltpu.async_copy(m2b, usp.at[rbuf], sem_s2, add=True)
            h1.wait()
            h2.wait()
            return carry

        lax.fori_loop(0, my_chunks, body, 0)
        plsc.subcore_barrier()
        for half in range(2):
            hseg = pl.ds(s * sl + half * hw, hw)
            oseg = pl.ds(c * npad + s * sl + half * hw, hw)
            pltpu.sync_copy(ssp.at[hseg], tmp)
            pltpu.sync_copy(tmp, s_out.at[oseg])
            pltpu.sync_copy(usp.at[hseg], tmp)
            pltpu.sync_copy(tmp, u_out.at[oseg])

    return k(row1d, col1d, ew1d, packed_pad)


def _tc_prep(degp, xp, *, n, npad):
    rows = npad // _LANES

    def body(dp_ref, xp_ref, out_ref, pk_ref):
        deg = dp_ref[0] + dp_ref[1] + 1.0
        dinv = lax.rsqrt(deg)
        i0 = lax.broadcasted_iota(jnp.int32, (rows, _LANES), 0)
        i1 = lax.broadcasted_iota(jnp.int32, (rows, _LANES), 1)
        valid = (i0 * _LANES + i1) < n
        dinv = jnp.where(valid, dinv, 0.0)
        a = dinv * xp_ref[...]
        out_ref[0] = dinv
        out_ref[1] = a
        rb_a = lax.bitcast_convert_type(a, jnp.uint32) + jnp.uint32(0x8000)
        rb_d = lax.bitcast_convert_type(dinv, jnp.uint32) + jnp.uint32(0x8000)
        packed = (rb_a & jnp.uint32(0xFFFF0000)) | (rb_d >> jnp.uint32(16))
        pk_ref[...] = lax.bitcast_convert_type(packed, jnp.int32)

    return pl.pallas_call(
        body,
        out_shape=(
            jax.ShapeDtypeStruct((2, rows, _LANES), jnp.float32),
            jax.ShapeDtypeStruct((rows, _LANES), jnp.int32),
        ),
    )(degp, xp)


def _tc_tail(sp, up, da, W1, b1, W2, b2, *, n, npad):
    rows = npad // _LANES
    f1 = W1.shape[1]
    f2 = W2.shape[1]

    def body(sp_ref, up_ref, da_ref, w1_ref, b1_ref, w2_ref, b2_ref, out_ref):
        dinv = da_ref[0]
        a = da_ref[1]
        s1 = dinv * (sp_ref[0] + sp_ref[1] + a)
        t = dinv * (up_ref[0] + up_ref[1] + dinv)
        vs = []
        for j in range(f1):
            hj = jnp.maximum(s1 * w1_ref[0, j] + b1_ref[j], 0.0)
            vs.append(jnp.sum(t * hj))
        zs = []
        for kk in range(f2):
            zs.append(sum(vs[j] * w2_ref[j, kk] for j in range(f1))
                      + float(n) * b2_ref[kk])
        m = zs[0]
        for kk in range(1, f2):
            m = jnp.maximum(m, zs[kk])
        es = [jnp.exp(z - m) for z in zs]
        tot = es[0]
        for kk in range(1, f2):
            tot = tot + es[kk]
        inv = 1.0 / tot
        colv = lax.broadcasted_iota(jnp.int32, (1, f2), 1)
        acc = jnp.where(colv == 0, es[0] * inv, 0.0)
        for kk in range(1, f2):
            acc = jnp.where(colv == kk, es[kk] * inv, acc)
        out_ref[...] = acc

    smemspec = pl.BlockSpec(memory_space=pltpu.SMEM)
    return pl.pallas_call(
        body,
        in_specs=[
            pl.BlockSpec((2, rows, _LANES), lambda: (0, 0, 0)),
            pl.BlockSpec((2, rows, _LANES), lambda: (0, 0, 0)),
            pl.BlockSpec((2, rows, _LANES), lambda: (0, 0, 0)),
            smemspec, smemspec, smemspec, smemspec,
        ],
        out_shape=jax.ShapeDtypeStruct((1, f2), jnp.float32),
    )(sp, up, da, W1, b1, W2, b2)


def kernel(x, edge_index, edge_attr, W1, b1, W2, b2):
    n = x.shape[0]
    e = edge_index.shape[1]
    npad = ((n + _LANES - 1) // _LANES) * _LANES

    row1d = edge_index[0]
    col1d = edge_index[1]
    xp = jnp.pad(x[:, 0], (0, npad - n)).reshape(npad // _LANES, _LANES)

    ch1 = _pick_ch(e, 25600)
    ch2 = _pick_ch(e, 2560)

    degp = _sc_degree(col1d, ew1d := edge_attr, npad=npad, e=e, ch=ch1)
    da, packed = _tc_prep(degp.reshape(2, npad // _LANES, _LANES), xp,
                          n=n, npad=npad)
    s_p, u_p = _sc_messages(row1d, col1d, ew1d, packed.reshape(npad),
                            npad=npad, e=e, ch=ch2)
    out = _tc_tail(s_p.reshape(2, npad // _LANES, _LANES),
                   u_p.reshape(2, npad // _LANES, _LANES),
                   da, W1, b1, W2, b2, n=n, npad=npad)
    return out
